# baseline (speedup 1.0000x reference)
import functools

import jax
import jax.numpy as jnp
from jax import lax
from jax.experimental import pallas as pl
from jax.experimental.pallas import tpu as pltpu
from jax.experimental.pallas import tpu_sc as plsc

_NUM_CORES = 2
_NUM_SUBCORES = 16


def kernel(x, W):
    B, S = x.shape
    _, D = W.shape
    NW = _NUM_CORES * _NUM_SUBCORES
    rows_per_w = S // NW

    mesh = plsc.VectorSubcoreMesh(
        core_axis_name="c", subcore_axis_name="s", num_cores=_NUM_CORES
    )

    @functools.partial(
        pl.kernel,
        out_type=jax.ShapeDtypeStruct((B, S, D), jnp.float32),
        mesh=mesh,
        scratch_types=[pltpu.SemaphoreType.DMA],
    )
    def sc_copy(w_hbm, out_hbm, sem):
        wid = lax.axis_index("s") * _NUM_CORES + lax.axis_index("c")
        base = wid * rows_per_w
        copies = [
            pltpu.async_copy(
                w_hbm.at[pl.ds(base, rows_per_w)],
                out_hbm.at[b, pl.ds(base, rows_per_w)],
                sem,
            )
            for b in range(B)
        ]
        for c in copies:
            c.wait()

    return sc_copy(W)


# speedup vs baseline: 51.0896x; 51.0896x over previous
import functools

import jax
import jax.numpy as jnp
from jax import lax
from jax.experimental import pallas as pl
from jax.experimental.pallas import tpu as pltpu
from jax.experimental.pallas import tpu_sc as plsc

_NUM_CORES = 2
_NUM_SUBCORES = 16
_NBUF = 4
_CH = 16


def kernel(x, W):
    B, S = x.shape
    _, D = W.shape
    NW = _NUM_CORES * _NUM_SUBCORES
    rows_per_w = S // NW
    n_ch = rows_per_w // _CH

    mesh = plsc.VectorSubcoreMesh(
        core_axis_name="c", subcore_axis_name="s", num_cores=_NUM_CORES
    )

    @functools.partial(
        pl.kernel,
        out_type=jax.ShapeDtypeStruct((B, S, D), jnp.float32),
        mesh=mesh,
        scratch_types=[
            pltpu.VMEM((_NBUF, _CH, D), jnp.float32),
            pltpu.SemaphoreType.DMA((_NBUF,)),
            pltpu.SemaphoreType.DMA((_NBUF,)),
        ],
    )
    def sc_copy(w_hbm, out_hbm, buf, rsem, wsem):
        wid = lax.axis_index("s") * _NUM_CORES + lax.axis_index("c")
        base0 = wid * rows_per_w

        def rstart(k):
            return pltpu.async_copy(
                w_hbm.at[pl.ds(base0 + k * _CH, _CH)],
                buf.at[k % _NBUF],
                rsem.at[k % _NBUF],
            )

        def wstart(k):
            return [
                pltpu.async_copy(
                    buf.at[k % _NBUF],
                    out_hbm.at[b, pl.ds(base0 + k * _CH, _CH)],
                    wsem.at[k % _NBUF],
                )
                for b in range(B)
            ]

        reads, writes = {}, {}
        for j in range(min(_NBUF, n_ch)):
            reads[j] = rstart(j)
        for k in range(n_ch):
            reads[k].wait()
            writes[k] = wstart(k)
            nxt = k + 1
            if _NBUF <= nxt < n_ch:
                for c in writes[nxt - _NBUF]:
                    c.wait()
                reads[nxt] = rstart(nxt)
        for k in range(max(0, n_ch - _NBUF), n_ch):
            for c in writes[k]:
                c.wait()

    return sc_copy(W)


# speedup vs baseline: 54.7553x; 1.0717x over previous
import functools

import jax
import jax.numpy as jnp
from jax import lax
from jax.experimental import pallas as pl
from jax.experimental.pallas import tpu as pltpu
from jax.experimental.pallas import tpu_sc as plsc

_NUM_CORES = 2
_NUM_SUBCORES = 16
_NBUF = 3
_CH = 32


def kernel(x, W):
    B, S = x.shape
    _, D = W.shape
    NW = _NUM_CORES * _NUM_SUBCORES
    rows_per_w = S // NW
    n_ch = rows_per_w // _CH

    mesh = plsc.VectorSubcoreMesh(
        core_axis_name="c", subcore_axis_name="s", num_cores=_NUM_CORES
    )

    @functools.partial(
        pl.kernel,
        out_type=jax.ShapeDtypeStruct((B, S, D), jnp.float32),
        mesh=mesh,
        scratch_types=[
            pltpu.VMEM((_NBUF, _CH, D), jnp.float32),
            pltpu.SemaphoreType.DMA((_NBUF,)),
            pltpu.SemaphoreType.DMA((_NBUF,)),
        ],
    )
    def sc_copy(w_hbm, out_hbm, buf, rsem, wsem):
        wid = lax.axis_index("s") * _NUM_CORES + lax.axis_index("c")
        base0 = wid * rows_per_w

        def rstart(k):
            return pltpu.async_copy(
                w_hbm.at[pl.ds(base0 + k * _CH, _CH)],
                buf.at[k % _NBUF],
                rsem.at[k % _NBUF],
            )

        def wstart(k):
            return [
                pltpu.async_copy(
                    buf.at[k % _NBUF],
                    out_hbm.at[b, pl.ds(base0 + k * _CH, _CH)],
                    wsem.at[k % _NBUF],
                )
                for b in range(B)
            ]

        reads, writes = {}, {}
        for j in range(min(_NBUF, n_ch)):
            reads[j] = rstart(j)
        for k in range(n_ch):
            reads[k].wait()
            writes[k] = wstart(k)
            nxt = k + 1
            if _NBUF <= nxt < n_ch:
                for c in writes[nxt - _NBUF]:
                    c.wait()
                reads[nxt] = rstart(nxt)
        for k in range(max(0, n_ch - _NBUF), n_ch):
            for c in writes[k]:
                c.wait()

    return sc_copy(W)
